# baseline (speedup 1.0000x reference)
import functools

import jax
import jax.numpy as jnp
from jax.experimental import pallas as pl

N_EXPERTS = 64
K_ACTIVE = 8
BT = 512


def _router_kernel(x_ref, wt_ref, topi_ref, w_ref):
    logits = jax.lax.dot_general(
        x_ref[...], wt_ref[...],
        dimension_numbers=(((1,), (0,)), ((), ())),
        preferred_element_type=jnp.float32,
    )

    lanes = jax.lax.broadcasted_iota(jnp.int32, logits.shape, 1)
    neg_inf = jnp.float32(-jnp.inf)

    vals = logits
    top_vs = []
    top_is = []
    for _ in range(K_ACTIVE):
        m = jnp.max(vals, axis=-1, keepdims=True)
        idx = jnp.min(jnp.where(vals == m, lanes, N_EXPERTS), axis=-1,
                      keepdims=True)
        top_vs.append(m)
        top_is.append(idx)
        vals = jnp.where(lanes == idx, neg_inf, vals)

    topv = jnp.concatenate(top_vs, axis=-1)
    topi = jnp.concatenate(top_is, axis=-1)

    e = jnp.exp(topv - topv[:, :1])
    w = e / jnp.sum(e, axis=-1, keepdims=True)

    topi_ref[...] = topi
    w_ref[...] = w


@jax.jit
def kernel(x, W):
    n_tokens, d_model = x.shape
    wt = W.T
    grid = (n_tokens // BT,)
    topi, w = pl.pallas_call(
        _router_kernel,
        grid=grid,
        in_specs=[
            pl.BlockSpec((BT, d_model), lambda i: (i, 0)),
            pl.BlockSpec((d_model, N_EXPERTS), lambda i: (0, 0)),
        ],
        out_specs=[
            pl.BlockSpec((BT, K_ACTIVE), lambda i: (i, 0)),
            pl.BlockSpec((BT, K_ACTIVE), lambda i: (i, 0)),
        ],
        out_shape=[
            jax.ShapeDtypeStruct((n_tokens, K_ACTIVE), jnp.int32),
            jax.ShapeDtypeStruct((n_tokens, K_ACTIVE), jnp.float32),
        ],
    )(x, wt)
    return topi, w


# speedup vs baseline: 1.1528x; 1.1528x over previous
import jax
import jax.numpy as jnp
from jax.experimental import pallas as pl

N_EXPERTS = 64
K_ACTIVE = 8
BT = 512
NSPLIT = 4
D_CHUNK_HINT = None


def _router_kernel(*refs):
    x_refs = refs[:NSPLIT]
    wt_ref = refs[NSPLIT]
    topi_ref, w_ref = refs[NSPLIT + 1], refs[NSPLIT + 2]

    d_chunk = x_refs[0].shape[1]
    logits = jnp.zeros((x_refs[0].shape[0], N_EXPERTS), jnp.float32)
    for s in range(NSPLIT):
        logits += jax.lax.dot_general(
            x_refs[s][...], wt_ref[pl.ds(s * d_chunk, d_chunk), :],
            dimension_numbers=(((1,), (0,)), ((), ())),
            preferred_element_type=jnp.float32,
        )

    lanes = jax.lax.broadcasted_iota(
        jnp.int32, logits.shape, 1).astype(jnp.float32)
    neg_inf = jnp.float32(-jnp.inf)
    big_lane = jnp.float32(2.0 * N_EXPERTS)

    vals = logits
    top_vs = []
    top_is = []
    for _ in range(K_ACTIVE):
        m = jnp.max(vals, axis=-1, keepdims=True)
        idx = jnp.min(jnp.where(vals == m, lanes, big_lane), axis=-1,
                      keepdims=True)
        top_vs.append(m)
        top_is.append(idx)
        vals = jnp.where(lanes == idx, neg_inf, vals)

    topv = jnp.concatenate(top_vs, axis=-1)
    topi = jnp.concatenate(top_is, axis=-1).astype(jnp.int32)

    e = jnp.exp(topv - topv[:, :1])
    w = e / jnp.sum(e, axis=-1, keepdims=True)

    topi_ref[...] = topi
    w_ref[...] = w


@jax.jit
def kernel(x, W):
    n_tokens, d_model = x.shape
    wt = W.T
    d_chunk = d_model // NSPLIT
    grid = (n_tokens // BT,)
    x_specs = [
        pl.BlockSpec((BT, d_chunk), lambda i, s=s: (i, s))
        for s in range(NSPLIT)
    ]
    topi, w = pl.pallas_call(
        _router_kernel,
        grid=grid,
        in_specs=x_specs + [
            pl.BlockSpec((d_model, N_EXPERTS), lambda i: (0, 0)),
        ],
        out_specs=[
            pl.BlockSpec((BT, K_ACTIVE), lambda i: (i, 0)),
            pl.BlockSpec((BT, K_ACTIVE), lambda i: (i, 0)),
        ],
        out_shape=[
            jax.ShapeDtypeStruct((n_tokens, K_ACTIVE), jnp.int32),
            jax.ShapeDtypeStruct((n_tokens, K_ACTIVE), jnp.float32),
        ],
    )(*([x] * NSPLIT), wt)
    return topi, w
